# Initial kernel scaffold; baseline (speedup 1.0000x reference)
import jax
import jax.numpy as jnp
from jax.experimental import pallas as pl


def kernel(tokens, segments, token_table, segment_table, pos_weight):
    raise NotImplementedError("write your pallas kernel here")


# speedup vs baseline: 3.3236x; 3.3236x over previous
import functools

import jax
import jax.numpy as jnp
from jax import lax
from jax.experimental import pallas as pl
from jax.experimental.pallas import tpu as pltpu
from jax.experimental.pallas import tpu_sc as plsc

VOCAB = 100000
HIDDEN = 128
MAXLEN = 512
BATCH = 256

NC, NS = 2, 16
NW = NC * NS
ROWS = BATCH * MAXLEN
RPW = ROWS // NW
CH = 128
NCHUNK = RPW // CH


def _prep_body(seg_tab_ref, pos_ref, segs_ref, comb_ref, cidx_ref):
    comb_ref[...] = seg_tab_ref[...][:, None, :] + pos_ref[...][None, :, :]
    pos_ids = lax.broadcasted_iota(jnp.int32, (BATCH, MAXLEN), 1)
    cidx_ref[...] = segs_ref[...] * MAXLEN + pos_ids


def _prep(segment_table, pos_weight, segments):
    return pl.pallas_call(
        _prep_body,
        out_shape=(
            jax.ShapeDtypeStruct((2, MAXLEN, HIDDEN), jnp.float32),
            jax.ShapeDtypeStruct((BATCH, MAXLEN), jnp.int32),
        ),
    )(segment_table, pos_weight, segments)


@functools.partial(
    pl.kernel,
    out_type=jax.ShapeDtypeStruct((ROWS, HIDDEN), jnp.float32),
    mesh=plsc.VectorSubcoreMesh(core_axis_name="c", subcore_axis_name="s"),
    scratch_types=[
        pltpu.VMEM((NCHUNK, CH), jnp.int32),
        pltpu.VMEM((NCHUNK, CH), jnp.int32),
        pltpu.VMEM((CH, HIDDEN), jnp.float32),
        pltpu.SemaphoreType.DMA,
    ],
)
def _sc_embed(tok_hbm, cidx_hbm, table_hbm, comb_hbm, out_hbm,
              tki, cvi, buf, sem):
    wid = lax.axis_index("s") * NC + lax.axis_index("c")
    base = wid * RPW
    pltpu.sync_copy(tok_hbm.at[wid], tki)
    pltpu.sync_copy(cidx_hbm.at[wid], cvi)

    @pl.loop(0, NCHUNK)
    def _chunk(j):
        pltpu.async_copy(comb_hbm.at[cvi.at[j]], buf, sem).wait()
        pltpu.async_copy(table_hbm.at[tki.at[j]], buf, sem, add=True).wait()
        pltpu.sync_copy(buf, out_hbm.at[pl.ds(base + j * CH, CH)])


def kernel(tokens, segments, token_table, segment_table, pos_weight):
    comb, cidx = _prep(segment_table, pos_weight,
                       segments.astype(jnp.int32))
    comb = comb.reshape(2 * MAXLEN, HIDDEN)
    tok = tokens.astype(jnp.int32).reshape(NW, NCHUNK, CH)
    cidx = cidx.reshape(NW, NCHUNK, CH)
    out = _sc_embed(tok, cidx, token_table, comb)
    return out.reshape(BATCH, MAXLEN, HIDDEN)


# speedup vs baseline: 4.1416x; 1.2461x over previous
import functools

import jax
import jax.numpy as jnp
from jax import lax
from jax.experimental import pallas as pl
from jax.experimental.pallas import tpu as pltpu
from jax.experimental.pallas import tpu_sc as plsc

VOCAB = 100000
HIDDEN = 128
MAXLEN = 512
BATCH = 256

NC, NS = 2, 16
NW = NC * NS
ROWS = BATCH * MAXLEN
RPW = ROWS // NW
CH = 128
NCHUNK = RPW // CH


def _prep_body(seg_tab_ref, pos_ref, segs_ref, comb_ref, cidx_ref):
    comb_ref[...] = seg_tab_ref[...][:, None, :] + pos_ref[...][None, :, :]
    pos_ids = lax.broadcasted_iota(jnp.int32, (BATCH, MAXLEN), 1)
    cidx_ref[...] = segs_ref[...] * MAXLEN + pos_ids


def _prep(segment_table, pos_weight, segments):
    return pl.pallas_call(
        _prep_body,
        out_shape=(
            jax.ShapeDtypeStruct((2, MAXLEN, HIDDEN), jnp.float32),
            jax.ShapeDtypeStruct((BATCH, MAXLEN), jnp.int32),
        ),
    )(segment_table, pos_weight, segments)


@functools.partial(
    pl.kernel,
    out_type=jax.ShapeDtypeStruct((ROWS, HIDDEN), jnp.float32),
    mesh=plsc.VectorSubcoreMesh(core_axis_name="c", subcore_axis_name="s"),
    scratch_types=[
        pltpu.VMEM((NCHUNK, CH), jnp.int32),
        pltpu.VMEM((NCHUNK, CH), jnp.int32),
        pltpu.VMEM((CH, HIDDEN), jnp.float32),
        pltpu.VMEM((CH, HIDDEN), jnp.float32),
        pltpu.SemaphoreType.DMA,
        pltpu.SemaphoreType.DMA,
        pltpu.SemaphoreType.DMA,
        pltpu.SemaphoreType.DMA,
    ],
)
def _sc_embed(tok_hbm, cidx_hbm, table_hbm, comb_hbm, out_hbm,
              tki, cvi, buf_a, buf_b, sg_a, sg_b, sw_a, sw_b):
    wid = lax.axis_index("s") * NC + lax.axis_index("c")
    base = wid * RPW
    pltpu.sync_copy(tok_hbm.at[wid], tki)
    pltpu.sync_copy(cidx_hbm.at[wid], cvi)

    def out_at(j):
        return out_hbm.at[pl.ds(base + j * CH, CH)]

    def g_init(j, buf, sem):
        pltpu.async_copy(comb_hbm.at[cvi.at[j]], buf, sem)

    def g_init_wait(j, buf, sem):
        pltpu.make_async_copy(comb_hbm.at[cvi.at[j]], buf, sem).wait()

    def g_add(j, buf, sem):
        pltpu.async_copy(table_hbm.at[tki.at[j]], buf, sem, add=True)

    def g_add_wait(j, buf, sem):
        pltpu.make_async_copy(table_hbm.at[tki.at[j]], buf, sem).wait()

    def wr(j, buf, sem):
        pltpu.async_copy(buf, out_at(j), sem)

    def wr_wait(j, buf, sem):
        pltpu.make_async_copy(buf, out_at(j), sem).wait()

    g_init(0, buf_a, sg_a)

    @pl.loop(0, NCHUNK // 2)
    def _pair(jj):
        j = jj * 2

        @pl.when(jj > 0)
        def _():
            wr_wait(j - 1, buf_b, sw_b)
        g_init(j + 1, buf_b, sg_b)

        g_init_wait(j, buf_a, sg_a)
        g_add(j, buf_a, sg_a)
        g_add_wait(j, buf_a, sg_a)
        wr(j, buf_a, sw_a)

        g_init_wait(j + 1, buf_b, sg_b)
        g_add(j + 1, buf_b, sg_b)
        g_add_wait(j + 1, buf_b, sg_b)
        wr(j + 1, buf_b, sw_b)

        wr_wait(j, buf_a, sw_a)

        @pl.when(jj < NCHUNK // 2 - 1)
        def _():
            g_init(j + 2, buf_a, sg_a)

    wr_wait(NCHUNK - 1, buf_b, sw_b)


def kernel(tokens, segments, token_table, segment_table, pos_weight):
    comb, cidx = _prep(segment_table, pos_weight,
                       segments.astype(jnp.int32))
    comb = comb.reshape(2 * MAXLEN, HIDDEN)
    tok = tokens.astype(jnp.int32).reshape(NW, NCHUNK, CH)
    cidx = cidx.reshape(NW, NCHUNK, CH)
    out = _sc_embed(tok, cidx, token_table, comb)
    return out.reshape(BATCH, MAXLEN, HIDDEN)


# speedup vs baseline: 5.6227x; 1.3576x over previous
import functools

import jax
import jax.numpy as jnp
from jax import lax
from jax.experimental import pallas as pl
from jax.experimental.pallas import tpu as pltpu
from jax.experimental.pallas import tpu_sc as plsc

VOCAB = 100000
HIDDEN = 128
MAXLEN = 512
BATCH = 256

NC, NS = 2, 16
NW = NC * NS
ROWS = BATCH * MAXLEN
RPW = ROWS // NW
CH = 128
NCHUNK = RPW // CH


def _prep_body(seg_tab_ref, pos_ref, segs_ref, comb_ref, cidx_ref):
    comb_ref[...] = seg_tab_ref[...][:, None, :] + pos_ref[...][None, :, :]
    pos_ids = lax.broadcasted_iota(jnp.int32, (BATCH, MAXLEN), 1)
    cidx_ref[...] = segs_ref[...] * MAXLEN + pos_ids


def _prep(segment_table, pos_weight, segments):
    return pl.pallas_call(
        _prep_body,
        out_shape=(
            jax.ShapeDtypeStruct((2, MAXLEN, HIDDEN), jnp.float32),
            jax.ShapeDtypeStruct((BATCH, MAXLEN), jnp.int32),
        ),
    )(segment_table, pos_weight, segments)


@functools.partial(
    pl.kernel,
    out_type=jax.ShapeDtypeStruct((ROWS, HIDDEN), jnp.float32),
    mesh=plsc.VectorSubcoreMesh(core_axis_name="c", subcore_axis_name="s"),
    scratch_types=[
        pltpu.VMEM((NCHUNK, CH), jnp.int32),
        pltpu.VMEM((NCHUNK, CH), jnp.int32),
        pltpu.VMEM((CH, HIDDEN), jnp.float32),
        pltpu.VMEM((CH, HIDDEN), jnp.float32),
        pltpu.VMEM_SHARED((2 * MAXLEN, HIDDEN), jnp.float32),
        pltpu.SemaphoreType.DMA,
        pltpu.SemaphoreType.DMA,
        pltpu.SemaphoreType.DMA,
        pltpu.SemaphoreType.DMA,
    ],
)
def _sc_embed(tok_hbm, cidx_hbm, table_hbm, comb_hbm, out_hbm,
              tki, cvi, buf_a, buf_b, comb_sp, sg_a, sg_b, sw_a, sw_b):
    wid = lax.axis_index("s") * NC + lax.axis_index("c")
    base = wid * RPW

    @pl.when(lax.axis_index("s") == 0)
    def _fill_spmem():
        pltpu.sync_copy(comb_hbm, comb_sp)

    pltpu.sync_copy(tok_hbm.at[wid], tki)
    pltpu.sync_copy(cidx_hbm.at[wid], cvi)
    plsc.subcore_barrier()

    def out_at(j):
        return out_hbm.at[pl.ds(base + j * CH, CH)]

    def g_init(j, buf, sem):
        pltpu.async_copy(comb_sp.at[cvi.at[j]], buf, sem)

    def g_init_wait(j, buf, sem):
        pltpu.make_async_copy(comb_sp.at[cvi.at[j]], buf, sem).wait()

    def g_add(j, buf, sem):
        pltpu.async_copy(table_hbm.at[tki.at[j]], buf, sem, add=True)

    def g_add_wait(j, buf, sem):
        pltpu.make_async_copy(table_hbm.at[tki.at[j]], buf, sem).wait()

    def wr(j, buf, sem):
        pltpu.async_copy(buf, out_at(j), sem)

    def wr_wait(j, buf, sem):
        pltpu.make_async_copy(buf, out_at(j), sem).wait()

    g_init(0, buf_a, sg_a)

    @pl.loop(0, NCHUNK // 2)
    def _pair(jj):
        j = jj * 2

        @pl.when(jj > 0)
        def _():
            wr_wait(j - 1, buf_b, sw_b)
        g_init(j + 1, buf_b, sg_b)

        g_init_wait(j, buf_a, sg_a)
        g_add(j, buf_a, sg_a)
        g_add_wait(j, buf_a, sg_a)
        wr(j, buf_a, sw_a)

        g_init_wait(j + 1, buf_b, sg_b)
        g_add(j + 1, buf_b, sg_b)
        g_add_wait(j + 1, buf_b, sg_b)
        wr(j + 1, buf_b, sw_b)

        wr_wait(j, buf_a, sw_a)

        @pl.when(jj < NCHUNK // 2 - 1)
        def _():
            g_init(j + 2, buf_a, sg_a)

    wr_wait(NCHUNK - 1, buf_b, sw_b)


def kernel(tokens, segments, token_table, segment_table, pos_weight):
    comb, cidx = _prep(segment_table, pos_weight,
                       segments.astype(jnp.int32))
    comb = comb.reshape(2 * MAXLEN, HIDDEN)
    tok = tokens.astype(jnp.int32).reshape(NW, NCHUNK, CH)
    cidx = cidx.reshape(NW, NCHUNK, CH)
    out = _sc_embed(tok, cidx, token_table, comb)
    return out.reshape(BATCH, MAXLEN, HIDDEN)
